# baseline (speedup 1.0000x reference)
import numpy as np
import jax
import jax.numpy as jnp
from jax.experimental import pallas as pl
from jax.experimental.pallas import tpu as pltpu

SR = 44100
T = 88200
N1 = 294
N2 = 300
B = 32
C = 2
E = 4
NCHUNK = B // E
LN10 = float(np.log(10.0))


def _dft_consts():
    def dftm(n):
        jk = np.outer(np.arange(n), np.arange(n)) % n
        w = np.exp(-2j * np.pi * jk / n)
        return w.real.astype(np.float32), w.imag.astype(np.float32)

    f1re, f1im = dftm(N1)
    f2re, f2im = dftm(N2)
    jn = np.outer(np.arange(N1), np.arange(N2)) % T
    tw = np.exp(-2j * np.pi * jn / T)
    return (f1re, f1im, f2re, f2im,
            tw.real.astype(np.float32), tw.imag.astype(np.float32))


_F1RE, _F1IM, _F2RE, _F2IM, _TWRE, _TWIM = _dft_consts()
_TWARE = np.tile(_TWRE, (1, E * C))
_TWAIM = np.tile(_TWIM, (1, E * C))
_TWVRE = np.tile(np.concatenate([_TWRE, _TWRE], axis=0), (E, 1))
_TWVIM = np.tile(np.concatenate([_TWIM, _TWIM], axis=0), (E, 1))

_KLIN = np.arange(T)
_FOLD = np.minimum(_KLIN, T - _KLIN).astype(np.int32)
_SGN_K = (np.where(_KLIN <= T // 2, 1.0, -1.0)
          .astype(np.float32).reshape(N2, N1).T.copy())
_SGN_V = np.concatenate([_SGN_K, _SGN_K], axis=0)


def _kmat_v(v_rfft):
    m = jnp.take(v_rfft, _FOLD).reshape(N2, N1).T
    return jnp.concatenate([m, m], axis=0)


def _fwd_body(x_ref, f1re_ref, f1im_ref, f2re_ref, f2im_ref,
              tware_ref, twaim_ref, outr_ref, outi_ref):
    def mm(a, bb):
        return jax.lax.dot_general(
            a, bb, (((1,), (0,)), ((), ())),
            preferred_element_type=jnp.float32)

    a2 = jnp.concatenate(
        [x_ref[e, c] for e in range(E) for c in range(C)], axis=1)
    br = mm(f1re_ref[...], a2)
    bi = mm(f1im_ref[...], a2)
    cr = br * tware_ref[...] - bi * twaim_ref[...]
    ci = br * twaim_ref[...] + bi * tware_ref[...]
    crv = jnp.concatenate(
        [cr[:, j * N2:(j + 1) * N2] for j in range(E * C)], axis=0)
    civ = jnp.concatenate(
        [ci[:, j * N2:(j + 1) * N2] for j in range(E * C)], axis=0)
    f2re = f2re_ref[...]
    f2im = f2im_ref[...]
    yr = mm(crv, f2re) - mm(civ, f2im)
    yi = mm(crv, f2im) + mm(civ, f2re)
    for e in range(E):
        outr_ref[e] = yr[e * 2 * N1:(e + 1) * 2 * N1]
        outi_ref[e] = yi[e * 2 * N1:(e + 1) * 2 * N1]


def _filter_body(xr_ref, xi_ref, p_ref, lab_ref,
                 f1re_ref, f1im_ref, f2re_ref, f2im_ref,
                 twre_ref, twim_ref,
                 wl_ref, wm_ref, wh_ref, m2pf_ref, sgn_ref, noise_ref, t_ref,
                 yr_ref, yi_ref):
    b = pl.program_id(0)
    lab = lab_ref[b]
    xr = xr_ref[0]
    xi = xi_ref[0]

    def mm(a, bb):
        return jax.lax.dot_general(
            a, bb, (((1,), (0,)), ((), ())),
            preferred_element_type=jnp.float32)

    def fwd1(a):
        br = mm(f1re_ref[...], a)
        bi = mm(f1im_ref[...], a)
        cr = br * twre_ref[...] - bi * twim_ref[...]
        ci = br * twim_ref[...] + bi * twre_ref[...]
        f2re = f2re_ref[...]
        f2im = f2im_ref[...]
        return (mm(cr, f2re) - mm(ci, f2im),
                mm(cr, f2im) + mm(ci, f2re))

    def passthrough():
        yr_ref[0] = xr
        yi_ref[0] = xi

    def br_eq():
        gl = p_ref[b, 0]
        gm = p_ref[b, 1]
        gh = p_ref[b, 2]
        curve = (wl_ref[...] * gl + wm_ref[...] * gm + wh_ref[...] * gh)
        g = jnp.exp(curve * jnp.float32(LN10 / 20.0))
        yr_ref[0] = xr * g
        yi_ref[0] = xi * g

    def br_mbc():
        accr = jnp.zeros((2 * N1, N2), jnp.float32)
        acci = jnp.zeros((2 * N1, N2), jnp.float32)
        for i, w_ref in enumerate((wl_ref, wm_ref, wh_ref)):
            w = w_ref[...]
            sr = xr * w
            si = xi * w
            thr = p_ref[b, 4 + 2 * i]
            ratio = p_ref[b, 5 + 2 * i]
            gs = []
            for c in range(C):
                pr = sr[c * N1:(c + 1) * N1]
                pi = si[c * N1:(c + 1) * N1]
                ms = jnp.sum(pr * pr + pi * pi) * jnp.float32(1.0 / T / T)
                msm = jnp.full((N1, N2), ms, jnp.float32)
                rms = jnp.sqrt(msm + 1e-8)
                lvl = jnp.log(rms + 1e-8) * jnp.float32(20.0 / LN10)
                gdb = jnp.where(lvl > thr,
                                (thr - lvl) * (1.0 - 1.0 / ratio), 0.0)
                gs.append(jnp.exp(gdb * jnp.float32(LN10 / 20.0)))
            g2 = jnp.concatenate(gs, axis=0)
            accr = accr + sr * g2
            acci = acci + si * g2
        yr_ref[0] = accr
        yi_ref[0] = acci

    def br_gain():
        g = p_ref[b, 10]
        yr_ref[0] = g * xr
        yi_ref[0] = g * xi

    def br_img():
        w = p_ref[b, 12]
        a = jnp.float32(0.5) * (1.0 + w)
        d = jnp.float32(0.5) * (1.0 - w)
        for z_ref, z in ((yr_ref, xr), (yi_ref, xi)):
            top = z[:N1]
            bot = z[N1:]
            z_ref[0] = jnp.concatenate(
                [a * top + d * bot, d * top + a * bot], axis=0)

    def br_delay():
        d = p_ref[b, 13]
        wet = p_ref[b, 14]
        ang = m2pf_ref[...] * d
        hre = jnp.cos(ang)
        him = sgn_ref[...] * jnp.sin(ang)
        yr_ref[0] = (1.0 - wet) * xr + wet * (xr * hre - xi * him)
        yi_ref[0] = (1.0 - wet) * xi + wet * (xr * him + xi * hre)

    def br_rev():
        de = p_ref[b, 15]
        wet = p_ref[b, 16]
        ir = noise_ref[...] * jnp.exp((-t_ref[...]) / de)
        e = jnp.sum(ir * ir)
        den = jnp.sqrt(jnp.full((N1, N2), e, jnp.float32)) + 1e-6
        irr, iri = fwd1(ir / den)
        irr2 = jnp.concatenate([irr, irr], axis=0)
        iri2 = jnp.concatenate([iri, iri], axis=0)
        yr_ref[0] = (1.0 - wet) * xr + wet * (xr * irr2 - xi * iri2)
        yi_ref[0] = (1.0 - wet) * xi + wet * (xr * iri2 + xi * irr2)

    branches = (br_eq, passthrough, br_mbc, br_gain, passthrough, br_img,
                br_delay, br_rev)
    for i, br in enumerate(branches):
        pl.when(lab == i)(br)


def _inv_body(yr_ref, yi_ref, x_ref, p_ref, lab_ref,
              f1re_ref, f1im_ref, f2re_ref, f2im_ref,
              twvre_ref, twvim_ref, out_ref):
    g = pl.program_id(0)

    def mm(a, bb):
        return jax.lax.dot_general(
            a, bb, (((1,), (0,)), ((), ())),
            preferred_element_type=jnp.float32)

    yrv = jnp.concatenate([yr_ref[e] for e in range(E)], axis=0)
    yiv = jnp.concatenate([yi_ref[e] for e in range(E)], axis=0)
    f2re = f2re_ref[...]
    f2im = f2im_ref[...]
    dr = mm(yrv, f2re) + mm(yiv, f2im)
    di = mm(yiv, f2re) - mm(yrv, f2im)
    er = dr * twvre_ref[...] + di * twvim_ref[...]
    ei = di * twvre_ref[...] - dr * twvim_ref[...]
    erh = jnp.concatenate(
        [jnp.concatenate([er[e * 2 * N1:e * 2 * N1 + N1],
                          er[e * 2 * N1 + N1:(e + 1) * 2 * N1]], axis=1)
         for e in range(E)], axis=1)
    eih = jnp.concatenate(
        [jnp.concatenate([ei[e * 2 * N1:e * 2 * N1 + N1],
                          ei[e * 2 * N1 + N1:(e + 1) * 2 * N1]], axis=1)
         for e in range(E)], axis=1)
    y = (mm(f1re_ref[...], erh) + mm(f1im_ref[...], eih)) * jnp.float32(1.0 / T)

    for e in range(E):
        idx = g * E + e
        lab = lab_ref[idx]
        out_ref[e, 0] = y[:, e * 2 * N2:e * 2 * N2 + N2]
        out_ref[e, 1] = y[:, e * 2 * N2 + N2:(e + 1) * 2 * N2]

        def fix_dist(_e=e, _idx=idx):
            gg = p_ref[_idx, 3]
            for c in range(C):
                out_ref[_e, c] = jnp.tanh(gg * x_ref[_e, c])

        def fix_lim(_e=e, _idx=idx):
            thr = p_ref[_idx, 11]
            for c in range(C):
                out_ref[_e, c] = thr * jnp.tanh(x_ref[_e, c] / thr)

        pl.when(lab == 1)(fix_dist)
        pl.when(lab == 4)(fix_lim)


def kernel(x, nn_param, labels):
    freqs = jnp.fft.rfftfreq(T, 1.0 / SR)
    lf = jnp.log10(freqs + 1e-3)
    t1 = jax.nn.sigmoid((lf - np.log10(250.0)) * 8.0)
    t2 = jax.nn.sigmoid((lf - np.log10(4000.0)) * 8.0)
    w_low = 1.0 - t1
    w_high = t2
    w_mid = t1 * (1.0 - t2)
    wl_v = _kmat_v(w_low)
    wm_v = _kmat_v(w_mid)
    wh_v = _kmat_v(w_high)
    m2pf_v = _kmat_v((-2.0 * jnp.pi) * freqs)
    noise_n = jax.random.normal(jax.random.key(42), (T,),
                                dtype=jnp.float32).reshape(N1, N2)
    t_n = (jnp.arange(T, dtype=jnp.float32) / SR).reshape(N1, N2)

    p = nn_param

    def dn(v, lo, hi):
        return lo + v * (hi - lo)

    ptab = jnp.stack([
        dn(p[:, 0], -12.0, 12.0),
        dn(p[:, 1], -12.0, 12.0),
        dn(p[:, 2], -12.0, 12.0),
        10.0 ** (dn(p[:, 3], 0.0, 8.0) / 20.0),
        dn(p[:, 4], -30.0, -5.0),
        dn(p[:, 5], 1.5, 6.0),
        dn(p[:, 6], -30.0, -5.0),
        dn(p[:, 7], 1.5, 6.0),
        dn(p[:, 8], -30.0, -5.0),
        dn(p[:, 9], 1.5, 6.0),
        10.0 ** (dn(p[:, 10], 6.0, 12.0) / 20.0),
        10.0 ** (dn(p[:, 11], -20.0, -1e-3) / 20.0),
        p[:, 12],
        dn(p[:, 13], 0.0, 300.0) / 1000.0,
        dn(p[:, 14], 0.1, 0.7),
        dn(p[:, 15], 0.05, 1.0) + 1e-4,
        dn(p[:, 16], 0.1, 0.7),
    ], axis=1).astype(jnp.float32)

    x4 = x.reshape(B, C, N1, N2)

    def cmat(a):
        nd = a.ndim
        return pl.BlockSpec(a.shape, lambda b, _n=nd: (0,) * _n)

    f1re = jnp.asarray(_F1RE)
    f1im = jnp.asarray(_F1IM)
    f2re = jnp.asarray(_F2RE)
    f2im = jnp.asarray(_F2IM)

    fwd_consts = (f1re, f1im, f2re, f2im,
                  jnp.asarray(_TWARE), jnp.asarray(_TWAIM))
    spec_shape = jax.ShapeDtypeStruct((B, 2 * N1, N2), jnp.float32)
    xr_all, xi_all = pl.pallas_call(
        _fwd_body,
        grid=(NCHUNK,),
        in_specs=[
            pl.BlockSpec((E, C, N1, N2), lambda gi: (gi, 0, 0, 0)),
        ] + [cmat(a) for a in fwd_consts],
        out_specs=[pl.BlockSpec((E, 2 * N1, N2), lambda gi: (gi, 0, 0))] * 2,
        out_shape=[spec_shape, spec_shape],
    )(x4, *fwd_consts)

    filt_consts = (f1re, f1im, f2re, f2im,
                   jnp.asarray(_TWRE), jnp.asarray(_TWIM),
                   wl_v, wm_v, wh_v, m2pf_v, jnp.asarray(_SGN_V),
                   noise_n, t_n)
    yr_all, yi_all = pl.pallas_call(
        _filter_body,
        grid=(B,),
        in_specs=[
            pl.BlockSpec((1, 2 * N1, N2), lambda b: (b, 0, 0)),
            pl.BlockSpec((1, 2 * N1, N2), lambda b: (b, 0, 0)),
            pl.BlockSpec(memory_space=pltpu.SMEM),
            pl.BlockSpec(memory_space=pltpu.SMEM),
        ] + [cmat(a) for a in filt_consts],
        out_specs=[pl.BlockSpec((1, 2 * N1, N2), lambda b: (b, 0, 0))] * 2,
        out_shape=[spec_shape, spec_shape],
    )(xr_all, xi_all, ptab, labels, *filt_consts)

    inv_consts = (f1re, f1im, f2re, f2im,
                  jnp.asarray(_TWVRE), jnp.asarray(_TWVIM))
    out4 = pl.pallas_call(
        _inv_body,
        grid=(NCHUNK,),
        in_specs=[
            pl.BlockSpec((E, 2 * N1, N2), lambda gi: (gi, 0, 0)),
            pl.BlockSpec((E, 2 * N1, N2), lambda gi: (gi, 0, 0)),
            pl.BlockSpec((E, C, N1, N2), lambda gi: (gi, 0, 0, 0)),
            pl.BlockSpec(memory_space=pltpu.SMEM),
            pl.BlockSpec(memory_space=pltpu.SMEM),
        ] + [cmat(a) for a in inv_consts],
        out_specs=pl.BlockSpec((E, C, N1, N2), lambda gi: (gi, 0, 0, 0)),
        out_shape=jax.ShapeDtypeStruct((B, C, N1, N2), jnp.float32),
    )(yr_all, yi_all, x4, ptab, labels, *inv_consts)

    out = out4.reshape(B, C, T)
    activate = jax.nn.one_hot(labels, 8, dtype=x.dtype)
    return (out, nn_param, activate, labels)


# speedup vs baseline: 1.1112x; 1.1112x over previous
import numpy as np
import jax
import jax.numpy as jnp
from jax.experimental import pallas as pl
from jax.experimental.pallas import tpu as pltpu

SR = 44100
T = 88200
N1 = 294
N2 = 300
B = 32
C = 2
LN10 = float(np.log(10.0))


def _dft_consts():
    def dftm(n):
        jk = np.outer(np.arange(n), np.arange(n)) % n
        w = np.exp(-2j * np.pi * jk / n)
        return w.real.astype(np.float32), w.imag.astype(np.float32)

    f1re, f1im = dftm(N1)
    f2re, f2im = dftm(N2)
    jn = np.outer(np.arange(N1), np.arange(N2)) % T
    tw = np.exp(-2j * np.pi * jn / T)
    return (f1re, f1im, f2re, f2im,
            tw.real.astype(np.float32), tw.imag.astype(np.float32))


_F1RE, _F1IM, _F2RE, _F2IM, _TWRE, _TWIM = _dft_consts()
_TW2RE = np.concatenate([_TWRE, _TWRE], axis=1)
_TW2IM = np.concatenate([_TWIM, _TWIM], axis=1)
_TWVRE = np.concatenate([_TWRE, _TWRE], axis=0)
_TWVIM = np.concatenate([_TWIM, _TWIM], axis=0)

_KLIN = np.arange(T)
_FOLD = np.minimum(_KLIN, T - _KLIN).astype(np.int32)
_SGN_K = (np.where(_KLIN <= T // 2, 1.0, -1.0)
          .astype(np.float32).reshape(N2, N1).T.copy())
_SGN_V = np.concatenate([_SGN_K, _SGN_K], axis=0)


def _kmat_v(v_rfft):
    m = jnp.take(v_rfft, _FOLD).reshape(N2, N1).T
    return jnp.concatenate([m, m], axis=0)


def _fx_body(x_ref, p_ref, lab_ref,
             f1re_ref, f1im_ref, f2re_ref, f2im_ref,
             twre_ref, twim_ref, tw2re_ref, tw2im_ref,
             twvre_ref, twvim_ref,
             wl_ref, wm_ref, wh_ref, m2pf_ref, sgn_ref, noise_ref, t_ref,
             out_ref):
    b = pl.program_id(0)
    lab = lab_ref[b]

    def mm(a, bb):
        return jax.lax.dot_general(
            a, bb, (((1,), (0,)), ((), ())),
            preferred_element_type=jnp.float32,
            precision=jax.lax.Precision.DEFAULT)

    def fwd2():
        a2 = jnp.concatenate([x_ref[0, 0], x_ref[0, 1]], axis=1)
        br = mm(f1re_ref[...], a2)
        bi = mm(f1im_ref[...], a2)
        cr = br * tw2re_ref[...] - bi * tw2im_ref[...]
        ci = br * tw2im_ref[...] + bi * tw2re_ref[...]
        crv = jnp.concatenate([cr[:, :N2], cr[:, N2:]], axis=0)
        civ = jnp.concatenate([ci[:, :N2], ci[:, N2:]], axis=0)
        f2re = f2re_ref[...]
        f2im = f2im_ref[...]
        return (mm(crv, f2re) - mm(civ, f2im),
                mm(crv, f2im) + mm(civ, f2re))

    def inv2(yr, yi):
        f2re = f2re_ref[...]
        f2im = f2im_ref[...]
        dr = mm(yr, f2re) + mm(yi, f2im)
        di = mm(yi, f2re) - mm(yr, f2im)
        er = dr * twvre_ref[...] + di * twvim_ref[...]
        ei = di * twvre_ref[...] - dr * twvim_ref[...]
        erh = jnp.concatenate([er[:N1], er[N1:]], axis=1)
        eih = jnp.concatenate([ei[:N1], ei[N1:]], axis=1)
        out = mm(f1re_ref[...], erh) + mm(f1im_ref[...], eih)
        return out * jnp.float32(1.0 / T)

    def fwd1(a):
        br = mm(f1re_ref[...], a)
        bi = mm(f1im_ref[...], a)
        cr = br * twre_ref[...] - bi * twim_ref[...]
        ci = br * twim_ref[...] + bi * twre_ref[...]
        f2re = f2re_ref[...]
        f2im = f2im_ref[...]
        return (mm(cr, f2re) - mm(ci, f2im),
                mm(cr, f2im) + mm(ci, f2re))

    def store2(res):
        out_ref[0, 0] = res[:, :N2]
        out_ref[0, 1] = res[:, N2:]

    def br_eq():
        gl = p_ref[b, 0]
        gm = p_ref[b, 1]
        gh = p_ref[b, 2]
        curve = (wl_ref[...] * gl + wm_ref[...] * gm + wh_ref[...] * gh)
        g = jnp.exp(curve * jnp.float32(LN10 / 20.0))
        xr, xi = fwd2()
        store2(inv2(xr * g, xi * g))

    def br_dist():
        g = p_ref[b, 3]
        for c in range(C):
            out_ref[0, c] = jnp.tanh(g * x_ref[0, c])

    def br_mbc():
        xr, xi = fwd2()
        accr = jnp.zeros((2 * N1, N2), jnp.float32)
        acci = jnp.zeros((2 * N1, N2), jnp.float32)
        for i, w_ref in enumerate((wl_ref, wm_ref, wh_ref)):
            w = w_ref[...]
            sr = xr * w
            si = xi * w
            thr = p_ref[b, 4 + 2 * i]
            ratio = p_ref[b, 5 + 2 * i]
            gs = []
            for c in range(C):
                pr = sr[c * N1:(c + 1) * N1]
                pi = si[c * N1:(c + 1) * N1]
                ms = jnp.sum(pr * pr + pi * pi) * jnp.float32(1.0 / T / T)
                msm = jnp.full((N1, N2), ms, jnp.float32)
                rms = jnp.sqrt(msm + 1e-8)
                lvl = jnp.log(rms + 1e-8) * jnp.float32(20.0 / LN10)
                gdb = jnp.where(lvl > thr,
                                (thr - lvl) * (1.0 - 1.0 / ratio), 0.0)
                gs.append(jnp.exp(gdb * jnp.float32(LN10 / 20.0)))
            g2 = jnp.concatenate(gs, axis=0)
            accr = accr + sr * g2
            acci = acci + si * g2
        store2(inv2(accr, acci))

    def br_gain():
        g = p_ref[b, 10]
        for c in range(C):
            out_ref[0, c] = g * x_ref[0, c]

    def br_lim():
        thr = p_ref[b, 11]
        for c in range(C):
            out_ref[0, c] = thr * jnp.tanh(x_ref[0, c] / thr)

    def br_img():
        w = p_ref[b, 12]
        x0 = x_ref[0, 0]
        x1 = x_ref[0, 1]
        mid = 0.5 * (x0 + x1)
        side = 0.5 * (x0 - x1)
        out_ref[0, 0] = mid + w * side
        out_ref[0, 1] = mid - w * side

    def br_delay():
        d = p_ref[b, 13]
        wet = p_ref[b, 14]
        ang = m2pf_ref[...] * d
        hre = jnp.cos(ang)
        him = sgn_ref[...] * jnp.sin(ang)
        xr, xi = fwd2()
        wet_sig = inv2(xr * hre - xi * him, xr * him + xi * hre)
        x2 = jnp.concatenate([x_ref[0, 0], x_ref[0, 1]], axis=1)
        store2((1.0 - wet) * x2 + wet * wet_sig)

    def br_rev():
        de = p_ref[b, 15]
        wet = p_ref[b, 16]
        ir = noise_ref[...] * jnp.exp((-t_ref[...]) / de)
        e = jnp.sum(ir * ir)
        den = jnp.sqrt(jnp.full((N1, N2), e, jnp.float32)) + 1e-6
        irr, iri = fwd1(ir / den)
        irr2 = jnp.concatenate([irr, irr], axis=0)
        iri2 = jnp.concatenate([iri, iri], axis=0)
        xr, xi = fwd2()
        wet_sig = inv2(xr * irr2 - xi * iri2, xr * iri2 + xi * irr2)
        x2 = jnp.concatenate([x_ref[0, 0], x_ref[0, 1]], axis=1)
        store2((1.0 - wet) * x2 + wet * wet_sig)

    branches = (br_eq, br_dist, br_mbc, br_gain, br_lim, br_img,
                br_delay, br_rev)
    for i, br in enumerate(branches):
        pl.when(lab == i)(br)


def kernel(x, nn_param, labels):
    freqs = jnp.fft.rfftfreq(T, 1.0 / SR)
    lf = jnp.log10(freqs + 1e-3)
    t1 = jax.nn.sigmoid((lf - np.log10(250.0)) * 8.0)
    t2 = jax.nn.sigmoid((lf - np.log10(4000.0)) * 8.0)
    w_low = 1.0 - t1
    w_high = t2
    w_mid = t1 * (1.0 - t2)
    wl_v = _kmat_v(w_low)
    wm_v = _kmat_v(w_mid)
    wh_v = _kmat_v(w_high)
    m2pf_v = _kmat_v((-2.0 * jnp.pi) * freqs)
    noise_n = jax.random.normal(jax.random.key(42), (T,),
                                dtype=jnp.float32).reshape(N1, N2)
    t_n = (jnp.arange(T, dtype=jnp.float32) / SR).reshape(N1, N2)

    p = nn_param

    def dn(v, lo, hi):
        return lo + v * (hi - lo)

    ptab = jnp.stack([
        dn(p[:, 0], -12.0, 12.0),
        dn(p[:, 1], -12.0, 12.0),
        dn(p[:, 2], -12.0, 12.0),
        10.0 ** (dn(p[:, 3], 0.0, 8.0) / 20.0),
        dn(p[:, 4], -30.0, -5.0),
        dn(p[:, 5], 1.5, 6.0),
        dn(p[:, 6], -30.0, -5.0),
        dn(p[:, 7], 1.5, 6.0),
        dn(p[:, 8], -30.0, -5.0),
        dn(p[:, 9], 1.5, 6.0),
        10.0 ** (dn(p[:, 10], 6.0, 12.0) / 20.0),
        10.0 ** (dn(p[:, 11], -20.0, -1e-3) / 20.0),
        p[:, 12],
        dn(p[:, 13], 0.0, 300.0) / 1000.0,
        dn(p[:, 14], 0.1, 0.7),
        dn(p[:, 15], 0.05, 1.0) + 1e-4,
        dn(p[:, 16], 0.1, 0.7),
    ], axis=1).astype(jnp.float32)

    x4 = x.reshape(B, C, N1, N2)

    def cmat(a):
        nd = a.ndim
        return pl.BlockSpec(a.shape, lambda b, _n=nd: (0,) * _n)

    consts = (jnp.asarray(_F1RE), jnp.asarray(_F1IM),
              jnp.asarray(_F2RE), jnp.asarray(_F2IM),
              jnp.asarray(_TWRE), jnp.asarray(_TWIM),
              jnp.asarray(_TW2RE), jnp.asarray(_TW2IM),
              jnp.asarray(_TWVRE), jnp.asarray(_TWVIM),
              wl_v, wm_v, wh_v, m2pf_v, jnp.asarray(_SGN_V),
              noise_n, t_n)

    out4 = pl.pallas_call(
        _fx_body,
        grid=(B,),
        in_specs=[
            pl.BlockSpec((1, C, N1, N2), lambda b: (b, 0, 0, 0)),
            pl.BlockSpec(memory_space=pltpu.SMEM),
            pl.BlockSpec(memory_space=pltpu.SMEM),
        ] + [cmat(a) for a in consts],
        out_specs=pl.BlockSpec((1, C, N1, N2), lambda b: (b, 0, 0, 0)),
        out_shape=jax.ShapeDtypeStruct((B, C, N1, N2), jnp.float32),
    )(x4, ptab, labels, *consts)

    out = out4.reshape(B, C, T)
    activate = jax.nn.one_hot(labels, 8, dtype=x.dtype)
    return (out, nn_param, activate, labels)
